# Initial kernel scaffold; baseline (speedup 1.0000x reference)
import jax
import jax.numpy as jnp
from jax.experimental import pallas as pl


def kernel(feats, im_shape, conv_w, conv_b, score_w, score_b, delta_w, delta_b):
    raise NotImplementedError("write your pallas kernel here")


# speedup vs baseline: 11.4670x; 11.4670x over previous
import functools

import jax
import jax.numpy as jnp
import numpy as np
from jax.experimental import pallas as pl
from jax.experimental.pallas import tpu as pltpu

PRE_NMS_TOP_N = 6000
POST_NMS_TOP_N = 1000
NMS_THRESH = 0.7
MIN_SIZE = 0.1
STRIDE = 16.0
ANCHOR_SIZES = (32.0, 64.0, 128.0, 256.0, 512.0)
ASPECT_RATIOS = (0.5, 1.0, 2.0)
BBOX_CLIP = float(np.log(1000.0 / 16.0))

NPAD = 6144
ROWS = NPAD // 128
OROWS = 1024 // 128


def _conv(x, w, b, pad):
    out = jax.lax.conv_general_dilated(x, w, window_strides=(1, 1),
                                       padding=[(pad, pad), (pad, pad)],
                                       dimension_numbers=('NCHW', 'OIHW', 'NCHW'))
    return out + b[None, :, None, None]


def _make_anchors(H, W):
    base = []
    for s in ANCHOR_SIZES:
        for r in ASPECT_RATIOS:
            w = s / np.sqrt(r)
            h = s * np.sqrt(r)
            base.append([-w / 2.0, -h / 2.0, w / 2.0, h / 2.0])
    base = jnp.asarray(np.array(base, dtype=np.float32))
    shift_x = jnp.arange(W, dtype=jnp.float32) * STRIDE
    shift_y = jnp.arange(H, dtype=jnp.float32) * STRIDE
    sy, sx = jnp.meshgrid(shift_y, shift_x, indexing='ij')
    shifts = jnp.stack([sx, sy, sx, sy], axis=-1)
    anchors = shifts[:, :, None, :] + base[None, None, :, :]
    return anchors.reshape(-1, 4)


def _nms_kernel(sc_ref, x1_ref, y1_ref, x2_ref, y2_ref, ki_ref, mm_ref):
    x1 = x1_ref[...]
    y1 = y1_ref[...]
    x2 = x2_ref[...]
    y2 = y2_ref[...]
    areas = (x2 - x1) * (y2 - y1)
    ii = (jax.lax.broadcasted_iota(jnp.int32, (ROWS, 128), 0) * 128
          + jax.lax.broadcasted_iota(jnp.int32, (ROWS, 128), 1))
    oi = (jax.lax.broadcasted_iota(jnp.int32, (OROWS, 128), 0) * 128
          + jax.lax.broadcasted_iota(jnp.int32, (OROWS, 128), 1))

    def step(t, carry):
        sc, ki, mm = carry
        m = jnp.max(sc)
        idx = jnp.min(jnp.where(sc == m, ii, jnp.int32(NPAD)))
        best = ii == idx
        bx1 = jnp.sum(jnp.where(best, x1, 0.0))
        by1 = jnp.sum(jnp.where(best, y1, 0.0))
        bx2 = jnp.sum(jnp.where(best, x2, 0.0))
        by2 = jnp.sum(jnp.where(best, y2, 0.0))
        barea = jnp.sum(jnp.where(best, areas, 0.0))
        xx1 = jnp.maximum(bx1, x1)
        yy1 = jnp.maximum(by1, y1)
        xx2 = jnp.minimum(bx2, x2)
        yy2 = jnp.minimum(by2, y2)
        inter = jnp.maximum(xx2 - xx1, 0.0) * jnp.maximum(yy2 - yy1, 0.0)
        iou = inter / (barea + areas - inter + 1e-10)
        sc = jnp.where(iou > NMS_THRESH, -jnp.inf, sc)
        sc = jnp.where(best, -jnp.inf, sc)
        sel = oi == t
        ki = jnp.where(sel, idx, ki)
        mm = jnp.where(sel, m, mm)
        return (sc, ki, mm)

    carry = (sc_ref[...], jnp.zeros((OROWS, 128), jnp.int32),
             jnp.zeros((OROWS, 128), jnp.float32))
    sc, ki, mm = jax.lax.fori_loop(0, POST_NMS_TOP_N, step, carry)
    ki_ref[...] = ki
    mm_ref[...] = mm


def _run_nms(sc, x1, y1, x2, y2):
    return pl.pallas_call(
        _nms_kernel,
        out_shape=(jax.ShapeDtypeStruct((OROWS, 128), jnp.int32),
                   jax.ShapeDtypeStruct((OROWS, 128), jnp.float32)),
    )(sc, x1, y1, x2, y2)


def kernel(feats, im_shape, conv_w, conv_b, score_w, score_b, delta_w, delta_b):
    rpn_feat = jax.nn.relu(_conv(feats, conv_w, conv_b, 1))
    scores = _conv(rpn_feat, score_w, score_b, 0)
    deltas = _conv(rpn_feat, delta_w, delta_b, 0)
    H, W = feats.shape[2], feats.shape[3]
    anchors = _make_anchors(H, W)
    s = jnp.transpose(scores[0], (1, 2, 0)).reshape(-1)
    d = jnp.transpose(deltas[0], (1, 2, 0)).reshape(-1, 4)
    probs = jax.nn.sigmoid(s)
    k = min(PRE_NMS_TOP_N, probs.shape[0])
    topv, topi = jax.lax.top_k(probs, k)
    td = d[topi]
    ta = anchors[topi]
    aw = ta[:, 2] - ta[:, 0]
    ah = ta[:, 3] - ta[:, 1]
    acx = ta[:, 0] + 0.5 * aw
    acy = ta[:, 1] + 0.5 * ah
    dx, dy = td[:, 0], td[:, 1]
    dw = jnp.minimum(td[:, 2], BBOX_CLIP)
    dh = jnp.minimum(td[:, 3], BBOX_CLIP)
    pcx = dx * aw + acx
    pcy = dy * ah + acy
    pw = jnp.exp(dw) * aw
    ph = jnp.exp(dh) * ah
    h_img = im_shape[0, 0]
    w_img = im_shape[0, 1]
    x1 = jnp.clip(pcx - 0.5 * pw, 0.0, w_img)
    y1 = jnp.clip(pcy - 0.5 * ph, 0.0, h_img)
    x2 = jnp.clip(pcx + 0.5 * pw, 0.0, w_img)
    y2 = jnp.clip(pcy + 0.5 * ph, 0.0, h_img)
    valid = ((x2 - x1) >= MIN_SIZE) & ((y2 - y1) >= MIN_SIZE)
    sc = jnp.where(valid, topv, -jnp.inf)

    def padp(v, fill):
        return jnp.concatenate(
            [v, jnp.full((NPAD - k,), fill, v.dtype)]).reshape(ROWS, 128)

    ki, _mm = _run_nms(padp(sc, -jnp.inf), padp(x1, 0.0),
                       padp(y1, 0.0), padp(x2, 0.0), padp(y2, 0.0))
    keep = ki.reshape(-1)[:POST_NMS_TOP_N]
    rois = jnp.stack([x1, y1, x2, y2], axis=-1)[keep]
    rois_num = jnp.array([POST_NMS_TOP_N], dtype=jnp.int32)
    return rois, rois_num


# speedup vs baseline: 13.8724x; 1.2098x over previous
import functools

import jax
import jax.numpy as jnp
import numpy as np
from jax.experimental import pallas as pl
from jax.experimental.pallas import tpu as pltpu

PRE_NMS_TOP_N = 6000
POST_NMS_TOP_N = 1000
NMS_THRESH = 0.7
MIN_SIZE = 0.1
STRIDE = 16.0
ANCHOR_SIZES = (32.0, 64.0, 128.0, 256.0, 512.0)
ASPECT_RATIOS = (0.5, 1.0, 2.0)
BBOX_CLIP = float(np.log(1000.0 / 16.0))

NPAD = 6144
ROWS = NPAD // 128
OROWS = 1024 // 128


def _conv(x, w, b, pad):
    out = jax.lax.conv_general_dilated(x, w, window_strides=(1, 1),
                                       padding=[(pad, pad), (pad, pad)],
                                       dimension_numbers=('NCHW', 'OIHW', 'NCHW'))
    return out + b[None, :, None, None]


def _make_anchors(H, W):
    base = []
    for s in ANCHOR_SIZES:
        for r in ASPECT_RATIOS:
            w = s / np.sqrt(r)
            h = s * np.sqrt(r)
            base.append([-w / 2.0, -h / 2.0, w / 2.0, h / 2.0])
    base = jnp.asarray(np.array(base, dtype=np.float32))
    shift_x = jnp.arange(W, dtype=jnp.float32) * STRIDE
    shift_y = jnp.arange(H, dtype=jnp.float32) * STRIDE
    sy, sx = jnp.meshgrid(shift_y, shift_x, indexing='ij')
    shifts = jnp.stack([sx, sy, sx, sy], axis=-1)
    anchors = shifts[:, :, None, :] + base[None, None, :, :]
    return anchors.reshape(-1, 4)


def _nms_kernel(sc_ref, x1_ref, y1_ref, x2_ref, y2_ref, ki_ref):
    x1 = x1_ref[...]
    y1 = y1_ref[...]
    x2 = x2_ref[...]
    y2 = y2_ref[...]
    areas = (x2 - x1) * (y2 - y1)
    ii = (jax.lax.broadcasted_iota(jnp.int32, (ROWS, 128), 0) * 128
          + jax.lax.broadcasted_iota(jnp.int32, (ROWS, 128), 1))
    oi = (jax.lax.broadcasted_iota(jnp.int32, (OROWS, 128), 0) * 128
          + jax.lax.broadcasted_iota(jnp.int32, (OROWS, 128), 1))

    def step(t, carry):
        iim, ki = carry
        idx = jnp.min(iim)
        best = ii == idx
        bx1 = jnp.sum(jnp.where(best, x1, 0.0))
        by1 = jnp.sum(jnp.where(best, y1, 0.0))
        bx2 = jnp.sum(jnp.where(best, x2, 0.0))
        by2 = jnp.sum(jnp.where(best, y2, 0.0))
        barea = (bx2 - bx1) * (by2 - by1)
        xx1 = jnp.maximum(bx1, x1)
        yy1 = jnp.maximum(by1, y1)
        xx2 = jnp.minimum(bx2, x2)
        yy2 = jnp.minimum(by2, y2)
        inter = jnp.maximum(xx2 - xx1, 0.0) * jnp.maximum(yy2 - yy1, 0.0)
        iou = inter / (barea + areas - inter + 1e-10)
        iim = jnp.where((iou > NMS_THRESH) | best, jnp.int32(NPAD), iim)
        ki = jnp.where(oi == t, jnp.where(idx == NPAD, 0, idx), ki)
        return (iim, ki)

    iim0 = jnp.where(sc_ref[...] != -jnp.inf, ii, jnp.int32(NPAD))
    carry = (iim0, jnp.zeros((OROWS, 128), jnp.int32))
    _, ki = jax.lax.fori_loop(0, POST_NMS_TOP_N, step, carry)
    ki_ref[...] = ki


def _run_nms(sc, x1, y1, x2, y2):
    return pl.pallas_call(
        _nms_kernel,
        out_shape=jax.ShapeDtypeStruct((OROWS, 128), jnp.int32),
    )(sc, x1, y1, x2, y2)


def kernel(feats, im_shape, conv_w, conv_b, score_w, score_b, delta_w, delta_b):
    rpn_feat = jax.nn.relu(_conv(feats, conv_w, conv_b, 1))
    scores = _conv(rpn_feat, score_w, score_b, 0)
    deltas = _conv(rpn_feat, delta_w, delta_b, 0)
    H, W = feats.shape[2], feats.shape[3]
    anchors = _make_anchors(H, W)
    s = jnp.transpose(scores[0], (1, 2, 0)).reshape(-1)
    d = jnp.transpose(deltas[0], (1, 2, 0)).reshape(-1, 4)
    probs = jax.nn.sigmoid(s)
    k = min(PRE_NMS_TOP_N, probs.shape[0])
    topv, topi = jax.lax.top_k(probs, k)
    td = d[topi]
    ta = anchors[topi]
    aw = ta[:, 2] - ta[:, 0]
    ah = ta[:, 3] - ta[:, 1]
    acx = ta[:, 0] + 0.5 * aw
    acy = ta[:, 1] + 0.5 * ah
    dx, dy = td[:, 0], td[:, 1]
    dw = jnp.minimum(td[:, 2], BBOX_CLIP)
    dh = jnp.minimum(td[:, 3], BBOX_CLIP)
    pcx = dx * aw + acx
    pcy = dy * ah + acy
    pw = jnp.exp(dw) * aw
    ph = jnp.exp(dh) * ah
    h_img = im_shape[0, 0]
    w_img = im_shape[0, 1]
    x1 = jnp.clip(pcx - 0.5 * pw, 0.0, w_img)
    y1 = jnp.clip(pcy - 0.5 * ph, 0.0, h_img)
    x2 = jnp.clip(pcx + 0.5 * pw, 0.0, w_img)
    y2 = jnp.clip(pcy + 0.5 * ph, 0.0, h_img)
    valid = ((x2 - x1) >= MIN_SIZE) & ((y2 - y1) >= MIN_SIZE)
    sc = jnp.where(valid, topv, -jnp.inf)

    def padp(v, fill):
        return jnp.concatenate(
            [v, jnp.full((NPAD - k,), fill, v.dtype)]).reshape(ROWS, 128)

    ki = _run_nms(padp(sc, -jnp.inf), padp(x1, 0.0),
                  padp(y1, 0.0), padp(x2, 0.0), padp(y2, 0.0))
    keep = ki.reshape(-1)[:POST_NMS_TOP_N]
    rois = jnp.stack([x1, y1, x2, y2], axis=-1)[keep]
    rois_num = jnp.array([POST_NMS_TOP_N], dtype=jnp.int32)
    return rois, rois_num
